# baseline (speedup 1.0000x reference)
import jax
import jax.numpy as jnp
from jax.experimental import pallas as pl

_EXPERTS = 64
_BLOCK_T = 4096


def _gate_block(x_ref, w_ref, y_ref, logits_ref):
    logits = jax.lax.dot_general(
        x_ref[...], w_ref[...], (((1,), (1,)), ((), ())),
        preferred_element_type=jnp.float32,
    )
    logits_ref[...] = logits
    m = jnp.max(logits, axis=1, keepdims=True)
    e = jnp.exp(logits - m)
    s = jnp.sum(e, axis=1, keepdims=True)
    col = jax.lax.broadcasted_iota(jnp.int32, logits.shape, 1).astype(
        jnp.float32)
    i1 = jnp.min(jnp.where(logits == m, col, jnp.float32(_EXPERTS)),
                 axis=1, keepdims=True)
    at1 = col == i1
    l2 = jnp.where(at1, jnp.float32(-jnp.inf), logits)
    m2 = jnp.max(l2, axis=1, keepdims=True)
    i2 = jnp.min(jnp.where(l2 == m2, col, jnp.float32(_EXPERTS)),
                 axis=1, keepdims=True)
    keep = at1 | (col == i2)
    y_ref[...] = jnp.where(keep, e / s, jnp.float32(0.0))


def kernel(x, W):
    n_tokens, k_dim = x.shape
    grid = (n_tokens // _BLOCK_T,)
    y, logits = pl.pallas_call(
        _gate_block,
        grid=grid,
        in_specs=[
            pl.BlockSpec((_BLOCK_T, k_dim), lambda i: (i, 0)),
            pl.BlockSpec(W.shape, lambda i: (0, 0)),
        ],
        out_specs=[
            pl.BlockSpec((_BLOCK_T, _EXPERTS), lambda i: (i, 0)),
            pl.BlockSpec((_BLOCK_T, _EXPERTS), lambda i: (i, 0)),
        ],
        out_shape=[
            jax.ShapeDtypeStruct((n_tokens, _EXPERTS), jnp.float32),
            jax.ShapeDtypeStruct((n_tokens, _EXPERTS), jnp.float32),
        ],
    )(x, W)
    return (y, logits)
